# baseline (speedup 1.0000x reference)
import functools

import jax
import jax.numpy as jnp
from jax.experimental import pallas as pl

_VOCAB = 1000
_EMB = 32
_T = 8
_ROWS_PER_TILE = 2048


def _head_kernel(idx_ref, emb_ref, pos_ref, w_ref, b_ref, out_ref):
    idx_col = idx_ref[:]
    iota = jax.lax.broadcasted_iota(jnp.int32, (idx_col.shape[0], _VOCAB), 1)
    onehot = (idx_col == iota).astype(jnp.float32)
    tok = jnp.dot(onehot, emb_ref[:], preferred_element_type=jnp.float32)
    x = tok + pos_ref[:]
    out_ref[:] = (
        jnp.dot(x, w_ref[:], preferred_element_type=jnp.float32) + b_ref[:]
    )


@jax.jit
def kernel(idx, embedding, positional_embedding, lm_head_w, lm_head_b):
    B, T = idx.shape
    n_rows = B * T
    R = _ROWS_PER_TILE
    grid = n_rows // R

    idx_col = idx.reshape(n_rows, 1).astype(jnp.int32)
    pos_tile = jnp.tile(positional_embedding, (R // T, 1))
    bias_row = lm_head_b.reshape(1, _VOCAB)

    out = pl.pallas_call(
        _head_kernel,
        grid=(grid,),
        in_specs=[
            pl.BlockSpec((R, 1), lambda i: (i, 0)),
            pl.BlockSpec((_VOCAB, _EMB), lambda i: (0, 0)),
            pl.BlockSpec((R, _EMB), lambda i: (0, 0)),
            pl.BlockSpec((_EMB, _VOCAB), lambda i: (0, 0)),
            pl.BlockSpec((1, _VOCAB), lambda i: (0, 0)),
        ],
        out_specs=pl.BlockSpec((R, _VOCAB), lambda i: (i, 0)),
        out_shape=jax.ShapeDtypeStruct((n_rows, _VOCAB), jnp.float32),
    )(idx_col, embedding, pos_tile, lm_head_w, bias_row)
    return out.reshape(B, T, _VOCAB)
